# baseline (speedup 1.0000x reference)
import functools

import jax
import jax.numpy as jnp
from jax import lax
from jax.experimental import pallas as pl
from jax.experimental.pallas import tpu as pltpu
from jax.experimental.pallas import tpu_sc as plsc

_NC, _NS = 2, 16
_NW = _NC * _NS
_BATCH = 4
_SEQ = 1024
_B = 4096
_D = 2048
_RPW = _B // _NW
_CHUNK = 8
_NBUF = 6
_NCHUNK = _RPW // _CHUNK

_mesh = plsc.VectorSubcoreMesh(core_axis_name="c", subcore_axis_name="s")


@functools.partial(
    pl.kernel,
    mesh=_mesh,
    out_type=jax.ShapeDtypeStruct((_B, _D), jnp.float32),
    scratch_types=[
        pltpu.VMEM((_RPW,), jnp.int32),
        pltpu.VMEM((_NBUF, _CHUNK, _D), jnp.float32),
        pltpu.SemaphoreType.DMA((_NBUF,)),
        pltpu.SemaphoreType.DMA((_NBUF,)),
    ],
)
def _sc_gather(idx_hbm, table_hbm, out_hbm, idx_v, rows_v, gsem, wsem):
    wid = lax.axis_index("s") * _NC + lax.axis_index("c")
    base = wid * _RPW
    pltpu.sync_copy(
        idx_hbm.at[wid // (_SEQ // _RPW), pl.ds((wid % (_SEQ // _RPW)) * _RPW, _RPW)],
        idx_v,
    )

    gathers = [None] * _NCHUNK
    writes = [None] * _NCHUNK
    writes2 = [None] * _NCHUNK

    def start_gather(g):
        b = g % _NBUF
        gathers[g] = pltpu.async_copy(
            table_hbm.at[idx_v.at[pl.ds(g * _CHUNK, _CHUNK)]],
            rows_v.at[b],
            gsem.at[b],
        )

    for g in range(_NBUF):
        start_gather(g)

    for g in range(_NCHUNK):
        b = g % _NBUF
        gathers[g].wait()
        writes[g] = pltpu.async_copy(
            rows_v.at[b, pl.ds(0, _CHUNK // 2)],
            out_hbm.at[pl.ds(base + g * _CHUNK, _CHUNK // 2)],
            wsem.at[b],
        )
        writes2[g] = pltpu.async_copy(
            rows_v.at[b, pl.ds(_CHUNK // 2, _CHUNK // 2)],
            out_hbm.at[pl.ds(base + g * _CHUNK + _CHUNK // 2, _CHUNK // 2)],
            wsem.at[b],
        )
        prev = g - 1
        if prev >= 0 and prev + _NBUF < _NCHUNK:
            writes[prev].wait()
            writes2[prev].wait()
            start_gather(prev + _NBUF)

    for g in range(_NCHUNK - _NBUF, _NCHUNK):
        writes[g].wait()
        writes2[g].wait()


def kernel(indices, table):
    out = _sc_gather(indices.astype(jnp.int32), table)
    return out.reshape(indices.shape + (table.shape[1],))


# speedup vs baseline: 1.0651x; 1.0651x over previous
import functools

import jax
import jax.numpy as jnp
from jax import lax
from jax.experimental import pallas as pl
from jax.experimental.pallas import tpu as pltpu
from jax.experimental.pallas import tpu_sc as plsc

_NC, _NS = 2, 16
_NW = _NC * _NS
_SEQ = 1024
_B = 4096
_D = 2048
_RPW = _B // _NW
_CHUNK = 8
_NBUF = 6
_NCHUNK = _RPW // _CHUNK

_mesh = plsc.VectorSubcoreMesh(core_axis_name="c", subcore_axis_name="s")


@functools.partial(
    pl.kernel,
    mesh=_mesh,
    out_type=jax.ShapeDtypeStruct((_B, _D), jnp.float32),
    scratch_types=[
        pltpu.VMEM((_RPW,), jnp.int32),
        pltpu.VMEM((_NBUF * _CHUNK, _D), jnp.float32),
        pltpu.SemaphoreType.DMA((_NBUF,)),
        pltpu.SemaphoreType.DMA((_NBUF,)),
    ],
)
def _sc_gather(idx_hbm, table_hbm, out_hbm, idx_v, rows_v, gsem, wsem):
    wid = lax.axis_index("s") * _NC + lax.axis_index("c")
    base = wid * _RPW
    pltpu.sync_copy(
        idx_hbm.at[wid // (_SEQ // _RPW), pl.ds((wid % (_SEQ // _RPW)) * _RPW, _RPW)],
        idx_v,
    )

    def gather_copy(g):
        b = lax.rem(g, _NBUF) if not isinstance(g, int) else g % _NBUF
        goff = pl.multiple_of(g * _CHUNK, 8)
        boff = pl.multiple_of(b * _CHUNK, 8)
        return pltpu.make_async_copy(
            table_hbm.at[idx_v.at[pl.ds(goff, _CHUNK)]],
            rows_v.at[pl.ds(boff, _CHUNK)],
            gsem.at[b],
        )

    def write_copy(g):
        b = lax.rem(g, _NBUF) if not isinstance(g, int) else g % _NBUF
        boff = pl.multiple_of(b * _CHUNK, 8)
        ooff = pl.multiple_of(base + g * _CHUNK, 8)
        return pltpu.make_async_copy(
            rows_v.at[pl.ds(boff, _CHUNK)],
            out_hbm.at[pl.ds(ooff, _CHUNK)],
            wsem.at[b],
        )

    for g in range(_NBUF):
        gather_copy(g).start()
    gather_copy(0).wait()
    write_copy(0).start()

    def body(g, carry):
        gather_copy(g).wait()
        write_copy(g).start()
        write_copy(g - 1).wait()
        gather_copy(g + _NBUF - 1).start()
        return carry

    lax.fori_loop(1, _NCHUNK - _NBUF + 1, body, 0)

    for g in range(_NCHUNK - _NBUF + 1, _NCHUNK):
        gather_copy(g).wait()
        write_copy(g).start()
        write_copy(g - 1).wait()
    write_copy(_NCHUNK - 1).wait()


def kernel(indices, table):
    out = _sc_gather(indices.astype(jnp.int32), table)
    return out.reshape(indices.shape + (table.shape[1],))


# speedup vs baseline: 1.0673x; 1.0021x over previous
import functools

import jax
import jax.numpy as jnp
from jax import lax
from jax.experimental import pallas as pl
from jax.experimental.pallas import tpu as pltpu
from jax.experimental.pallas import tpu_sc as plsc

_NC, _NS = 2, 16
_NW = _NC * _NS
_SEQ = 1024
_B = 4096
_D = 2048
_RPW = _B // _NW
_CHUNK = 8
_NBUF = 6
_NCHUNK = _RPW // _CHUNK

_mesh = plsc.VectorSubcoreMesh(core_axis_name="c", subcore_axis_name="s")


@functools.partial(
    pl.kernel,
    mesh=_mesh,
    out_type=jax.ShapeDtypeStruct((_B, _D), jnp.float32),
    scratch_types=[
        pltpu.VMEM((_RPW,), jnp.int32),
        pltpu.VMEM((_NBUF * _CHUNK, _D), jnp.float32),
        pltpu.SemaphoreType.DMA((_NBUF,)),
        pltpu.SemaphoreType.DMA((_NBUF,)),
    ],
)
def _sc_gather(idx_hbm, table_hbm, out_hbm, idx_v, rows_v, gsem, wsem):
    wid = lax.axis_index("s") * _NC + lax.axis_index("c")
    base = wid * _RPW
    pltpu.sync_copy(
        idx_hbm.at[wid // (_SEQ // _RPW), pl.ds((wid % (_SEQ // _RPW)) * _RPW, _RPW)],
        idx_v,
    )

    def gather_copy(g):
        b = lax.rem(g, _NBUF) if not isinstance(g, int) else g % _NBUF
        goff = pl.multiple_of(g * _CHUNK, 8)
        boff = pl.multiple_of(b * _CHUNK, 8)
        return pltpu.make_async_copy(
            table_hbm.at[idx_v.at[pl.ds(goff, _CHUNK)]],
            rows_v.at[pl.ds(boff, _CHUNK)],
            gsem.at[b],
        )

    def write_copy(g):
        b = lax.rem(g, _NBUF) if not isinstance(g, int) else g % _NBUF
        boff = pl.multiple_of(b * _CHUNK, 8)
        ooff = pl.multiple_of(base + g * _CHUNK, 8)
        return pltpu.make_async_copy(
            rows_v.at[pl.ds(boff, _CHUNK)],
            out_hbm.at[pl.ds(ooff, _CHUNK)],
            wsem.at[b],
        )

    def prime(g, carry):
        gather_copy(g).start()
        return carry

    lax.fori_loop(0, _NBUF, prime, 0)
    gather_copy(0).wait()
    write_copy(0).start()

    def body(g, carry):
        gather_copy(g).wait()
        write_copy(g).start()
        write_copy(g - 1).wait()
        gather_copy(g + _NBUF - 1).start()
        return carry

    lax.fori_loop(1, _NCHUNK - _NBUF + 1, body, 0)

    def tail(g, carry):
        gather_copy(g).wait()
        write_copy(g).start()
        write_copy(g - 1).wait()
        return carry

    lax.fori_loop(_NCHUNK - _NBUF + 1, _NCHUNK, tail, 0)
    write_copy(_NCHUNK - 1).wait()


def kernel(indices, table):
    out = _sc_gather(indices.astype(jnp.int32), table)
    return out.reshape(indices.shape + (table.shape[1],))
